# Initial kernel scaffold; baseline (speedup 1.0000x reference)
import jax
import jax.numpy as jnp
from jax.experimental import pallas as pl


def kernel(h, W0, W1):
    raise NotImplementedError("write your pallas kernel here")


# speedup vs baseline: 1.2655x; 1.2655x over previous
import functools

import jax
import jax.numpy as jnp
from jax import lax
from jax.experimental import pallas as pl
from jax.experimental.pallas import tpu as pltpu
from jax.experimental.pallas import tpu_sc as plsc

_B, _N, _DM = 4, 4096, 1024
_NS = 2
_SD = _DM // _NS
_K = 8192
_BETA = 0.25
_T = _B * _N

_TM = 512
_TN = 2736
_KPAD = 3 * _TN
_TB = _T // _TM
_KB = 3

_NW = 32
_TOK_W = _T // _NW
_CH = 128
_NCH = _TOK_W // _CH


def _argmin_body(xt_ref, w_ref, fs_ref, wsq_ref, ids_ref, mind_ref,
                 rmin_ref, ridx_ref):
    k = pl.program_id(2)
    mm = lax.dot_general(
        w_ref[0], xt_ref[0],
        dimension_numbers=(((1,), (0,)), ((), ())),
        preferred_element_type=jnp.float32)
    t1 = wsq_ref[:, :] + fs_ref[0]
    d = t1 - 2.0 * mm
    bmin = jnp.min(d, axis=0, keepdims=True)
    gidx = jax.lax.broadcasted_iota(jnp.int32, (_TN, _TM), 0) + k * _TN
    cand = jnp.where(d == bmin, gidx, jnp.int32(_KPAD))
    bidx = jnp.min(cand, axis=0, keepdims=True)

    is_first = k == 0
    is_last = k == pl.num_programs(2) - 1
    prev_min = rmin_ref[...]
    prev_idx = ridx_ref[...]
    take_new = jnp.logical_or(is_first, bmin < prev_min)
    val = jnp.where(take_new, bmin, prev_min)
    idx = jnp.where(take_new, bidx, prev_idx)
    rounded = val.astype(jnp.bfloat16).astype(jnp.float32)
    rmin_ref[...] = jnp.where(is_last, val, rounded)
    ridx_ref[...] = idx

    @pl.when(is_last)
    def _():
        ids_ref[0] = idx
        mind_ref[0] = val


def _argmin_call(xt, w_st, fs3, wsq2):
    return pl.pallas_call(
        _argmin_body,
        grid=(_NS, _TB, _KB),
        in_specs=[
            pl.BlockSpec((1, _SD, _TM), lambda s, t, k: (s, 0, t)),
            pl.BlockSpec((1, _TN, _SD), lambda s, t, k: (s, k, 0)),
            pl.BlockSpec((1, 1, _TM), lambda s, t, k: (s * _TB + t, 0, 0)),
            pl.BlockSpec((_TN, 1), lambda s, t, k: (s * _KB + k, 0)),
        ],
        out_specs=[
            pl.BlockSpec((1, 1, _TM), lambda s, t, k: (s * _TB + t, 0, 0)),
            pl.BlockSpec((1, 1, _TM), lambda s, t, k: (s * _TB + t, 0, 0)),
        ],
        out_shape=[
            jax.ShapeDtypeStruct((_NS * _TB, 1, _TM), jnp.int32),
            jax.ShapeDtypeStruct((_NS * _TB, 1, _TM), jnp.float32),
        ],
        compiler_params=pltpu.CompilerParams(
            dimension_semantics=("arbitrary", "arbitrary", "arbitrary")),
        scratch_shapes=[
            pltpu.VMEM((1, _TM), jnp.float32),
            pltpu.VMEM((1, _TM), jnp.int32),
        ],
    )(xt, w_st, fs3, wsq2)


def _gather_sc(w0, w1, ids0, ids1):
    mesh = plsc.VectorSubcoreMesh(core_axis_name="c", subcore_axis_name="s")

    @functools.partial(
        pl.kernel,
        mesh=mesh,
        out_type=[
            jax.ShapeDtypeStruct((_T, _SD), jnp.float32),
            jax.ShapeDtypeStruct((_T, _SD), jnp.float32),
        ],
        scratch_types=[
            pltpu.VMEM((_CH,), jnp.int32),
            pltpu.VMEM((_CH, _SD), jnp.float32),
            pltpu.SemaphoreType.DMA,
        ],
    )
    def gk(w0_hbm, w1_hbm, i0_hbm, i1_hbm, z0_hbm, z1_hbm, idx_v, rows_v, sem):
        wid = lax.axis_index("s") * 2 + lax.axis_index("c")
        base = wid * _TOK_W
        for w_hbm, i_hbm, z_hbm in ((w0_hbm, i0_hbm, z0_hbm),
                                    (w1_hbm, i1_hbm, z1_hbm)):
            for c in range(_NCH):
                off = base + c * _CH
                pltpu.sync_copy(i_hbm.at[pl.ds(off, _CH)], idx_v)
                pltpu.async_copy(w_hbm.at[idx_v], rows_v, sem).wait()
                pltpu.sync_copy(rows_v, z_hbm.at[pl.ds(off, _CH)])

    return gk(w0, w1, ids0, ids1)


def _st_body(h_ref, z0_ref, z1_ref, z_ref):
    h0 = h_ref[:, :_SD]
    h1 = h_ref[:, _SD:]
    z_ref[:, :_SD] = h0 + (z0_ref[...] - h0)
    z_ref[:, _SD:] = h1 + (z1_ref[...] - h1)


def _st_call(h2, z0, z1):
    tmz = 1024
    return pl.pallas_call(
        _st_body,
        grid=(_T // tmz,),
        in_specs=[
            pl.BlockSpec((tmz, _DM), lambda i: (i, 0)),
            pl.BlockSpec((tmz, _SD), lambda i: (i, 0)),
            pl.BlockSpec((tmz, _SD), lambda i: (i, 0)),
        ],
        out_specs=pl.BlockSpec((tmz, _DM), lambda i: (i, 0)),
        out_shape=jax.ShapeDtypeStruct((_T, _DM), jnp.float32),
    )(h2, z0, z1)


def kernel(h, W0, W1):
    hf = h.reshape(_T, _NS, _SD)
    xt = jnp.transpose(hf, (1, 2, 0))
    w_st = jnp.stack([W0, W1])
    w_st = jnp.pad(w_st, ((0, 0), (0, _KPAD - _K), (0, 0)))
    f0 = jnp.sum(hf[:, 0, :] ** 2, axis=1)
    f1 = jnp.sum(hf[:, 1, :] ** 2, axis=1)
    fs3 = jnp.stack([f0, f1]).reshape(_NS * _TB, 1, _TM)
    wsq = jnp.stack([jnp.sum(W0 ** 2, axis=1),
                     jnp.sum(W1 ** 2, axis=1)])
    wsq2 = jnp.pad(wsq, ((0, 0), (0, _KPAD - _K)),
                   constant_values=jnp.inf).reshape(_NS * _KPAD, 1)

    ids3, mind3 = _argmin_call(xt, w_st, fs3, wsq2)
    ids = ids3.reshape(_NS, _T)

    z0, z1 = _gather_sc(W0, W1, ids[0], ids[1])
    z = _st_call(h.reshape(_T, _DM), z0, z1).reshape(_B, _N, _DM)
    ids_packed = (ids[0] + _K * ids[1]).reshape(_B, _N)
    vq_total = (1.0 + _BETA) * (jnp.sum(mind3) / (_T * _SD))
    return (z, ids_packed, vq_total)
